# baseline (speedup 1.0000x reference)
import functools

import jax
import jax.numpy as jnp
from jax import lax
from jax.experimental import pallas as pl
from jax.experimental.pallas import tpu as pltpu
from jax.experimental.pallas import tpu_sc as plsc

_NC = 2
_NS = 16
_NW = _NC * _NS
_L = 16
_CH = 128
_G = 4


def _deg_kernel_body(n_pad, cpt, src_hbm, dst_hbm, ones_hbm, zb_hbm, out_hbm,
                     src_t, dst_t, ones_t, stage_t, din_sh, dout_sh):
    rows_per_tile = n_pad // _NS
    cid = lax.axis_index("c")
    sid = lax.axis_index("s")
    wid = cid * _NS + sid
    pltpu.sync_copy(src_hbm.at[wid], src_t)
    pltpu.sync_copy(dst_hbm.at[wid], dst_t)
    pltpu.sync_copy(ones_hbm, ones_t)
    pltpu.sync_copy(zb_hbm, stage_t)
    r0 = sid * rows_per_tile
    for jj in range(rows_per_tile // _CH):
        pltpu.sync_copy(stage_t, din_sh.at[pl.ds(r0 + jj * _CH, _CH)])
        pltpu.sync_copy(stage_t, dout_sh.at[pl.ds(r0 + jj * _CH, _CH)])
    plsc.subcore_barrier()

    @pl.loop(0, cpt)
    def _(j):
        pltpu.sync_copy(ones_t, din_sh.at[dst_t.at[j]], add=True)
        pltpu.sync_copy(ones_t, dout_sh.at[src_t.at[j]], add=True)

    plsc.subcore_barrier()
    for jj in range(rows_per_tile // _CH):
        sl = pl.ds(r0 + jj * _CH, _CH)
        pltpu.sync_copy(din_sh.at[sl], stage_t)
        pltpu.sync_copy(stage_t, out_hbm.at[cid, 0, sl])
        pltpu.sync_copy(dout_sh.at[sl], stage_t)
        pltpu.sync_copy(stage_t, out_hbm.at[cid, 1, sl])


def _prop_kernel_body(n_pad, cpt, dh, g_hbm, src_hbm, dst_hbm, zb_hbm, out_hbm,
                      sidx0, sidx1, didx0, didx1, rows0, rows1, rows2, rows3,
                      g_sh, acc_sh, gsem0, gsem1, gsem2, gsem3,
                      ssem0, ssem1, ssem2, ssem3, isem0, isem1):
    sidx = (sidx0, sidx1)
    didx = (didx0, didx1)
    rows = (rows0, rows1, rows2, rows3)
    gsem = (gsem0, gsem1, gsem2, gsem3)
    ssem = (ssem0, ssem1, ssem2, ssem3)
    isem = (isem0, isem1)
    rows_per_tile = n_pad // _NS
    cid = lax.axis_index("c")
    sid = lax.axis_index("s")
    r0 = sid * rows_per_tile
    ngrp = cpt // _G

    for jj in range(rows_per_tile // _CH):
        sl = pl.ds(r0 + jj * _CH, _CH)
        pltpu.sync_copy(g_hbm.at[cid, sl], rows0)
        pltpu.sync_copy(rows0, g_sh.at[sl])
    pltpu.sync_copy(zb_hbm, rows1)
    for jj in range(rows_per_tile // _CH):
        pltpu.sync_copy(rows1, acc_sh.at[pl.ds(r0 + jj * _CH, _CH)])
    plsc.subcore_barrier()

    def idx_issue(grp, slot):
        pltpu.async_copy(src_hbm.at[sid, pl.ds(grp * _G, _G)], sidx[slot],
                         isem[slot])
        pltpu.async_copy(dst_hbm.at[sid, pl.ds(grp * _G, _G)], didx[slot],
                         isem[slot])

    def idx_wait(slot):
        pltpu.make_async_copy(src_hbm.at[sid, pl.ds(0, _G)], sidx[slot],
                              isem[slot]).wait()
        pltpu.make_async_copy(dst_hbm.at[sid, pl.ds(0, _G)], didx[slot],
                              isem[slot]).wait()

    def process(slot):
        ds = [pltpu.async_copy(g_sh.at[sidx[slot].at[b]], rows[b], gsem[b])
              for b in range(_G)]
        ss = []
        for b in range(_G):
            ds[b].wait()
            ss.append(pltpu.async_copy(rows[b], acc_sh.at[didx[slot].at[b]],
                                       ssem[b], add=True))
        for b in range(_G):
            ss[b].wait()

    idx_issue(0, 0)
    idx_wait(0)
    idx_issue(1, 1)

    @pl.loop(0, ngrp // 2)
    def _(i2):
        g0 = 2 * i2
        process(0)
        idx_wait(1)

        @pl.when(g0 + 2 < ngrp)
        def _():
            idx_issue(g0 + 2, 0)

        process(1)

        @pl.when(g0 + 2 < ngrp)
        def _():
            idx_wait(0)

            @pl.when(g0 + 3 < ngrp)
            def _():
                idx_issue(g0 + 3, 1)

    plsc.subcore_barrier()
    for jj in range(rows_per_tile // _CH):
        sl = pl.ds(r0 + jj * _CH, _CH)
        pltpu.sync_copy(acc_sh.at[sl], rows0)
        pltpu.sync_copy(rows0, out_hbm.at[cid, sl])


def _combine0_body(feat_r, deg_r, w_r, b_r, g_r, out_r):
    dh = feat_r.shape[1] // 2
    dout = deg_r[0, 1, :, 0:1] + deg_r[1, 1, :, 0:1]
    inv_out = lax.rsqrt(jnp.maximum(dout, 1.0))
    h = feat_r[...]
    s = jnp.sum(h * w_r[:, 0][None, :], axis=1, keepdims=True) + b_r[0, 0]
    out_r[...] = s * h
    g = h * inv_out
    g_r[0] = g[:, :dh]
    g_r[1] = g[:, dh:]


def _combine_body(acc_r, deg_r, w_r, b_r, prev_r, g_r, out_r):
    din = deg_r[0, 0, :, 0:1] + deg_r[1, 0, :, 0:1]
    dout = deg_r[0, 1, :, 0:1] + deg_r[1, 1, :, 0:1]
    inv_in = lax.rsqrt(jnp.maximum(din, 1.0))
    inv_out = lax.rsqrt(jnp.maximum(dout, 1.0))
    h = jnp.concatenate([acc_r[0], acc_r[1]], axis=1) * inv_in
    s = jnp.sum(h * w_r[:, 0][None, :], axis=1, keepdims=True) + b_r[0, 0]
    out_r[...] = prev_r[...] + s * h
    g = h * inv_out
    dh = g.shape[1] // 2
    g_r[0] = g[:, :dh]
    g_r[1] = g[:, dh:]


def kernel(feat, edge_index, W, b):
    n, d = feat.shape
    dh = d // 2
    e = edge_index.shape[1]
    k_rounds = 10

    n_pad = -(-n // (_NS * _CH)) * (_NS * _CH)
    et = e + n
    cpt = -(-et // (_NS * _CH * 2 * _G)) * (2 * _G)
    e_pad = _NS * cpt * _CH

    idx_dtype = edge_index.dtype
    loop = jnp.arange(n, dtype=idx_dtype)
    pad = jnp.full((e_pad - et,), n, dtype=idx_dtype)
    src3 = jnp.concatenate([edge_index[0], loop, pad]).reshape(_NS, cpt, _CH)
    dst3 = jnp.concatenate([edge_index[1], loop, pad]).reshape(_NS, cpt, _CH)
    src_deg = src3.reshape(_NW, cpt // 2, _CH)
    dst_deg = dst3.reshape(_NW, cpt // 2, _CH)
    feat_pad = jnp.zeros((n_pad, d), feat.dtype).at[:n].set(feat)
    ones16 = jnp.ones((_CH, _L), jnp.float32)
    zb16 = jnp.zeros((_CH, _L), jnp.float32)
    zbd = jnp.zeros((_CH, dh), jnp.float32)
    b2 = b.reshape(1, 1)

    mesh = plsc.VectorSubcoreMesh(core_axis_name="c", subcore_axis_name="s")

    deg_call = functools.partial(
        pl.kernel,
        out_type=jax.ShapeDtypeStruct((_NC, 2, n_pad, _L), jnp.float32),
        mesh=mesh,
        scratch_types=[
            pltpu.VMEM((cpt // 2, _CH), jnp.int32),
            pltpu.VMEM((cpt // 2, _CH), jnp.int32),
            pltpu.VMEM((_CH, _L), jnp.float32),
            pltpu.VMEM((_CH, _L), jnp.float32),
            pltpu.VMEM_SHARED((n_pad, _L), jnp.float32),
            pltpu.VMEM_SHARED((n_pad, _L), jnp.float32),
        ],
        compiler_params=pltpu.CompilerParams(use_tc_tiling_on_sc=False),
    )(functools.partial(_deg_kernel_body, n_pad, cpt // 2))
    deg = deg_call(src_deg, dst_deg, ones16, zb16)

    prop_call = functools.partial(
        pl.kernel,
        out_type=jax.ShapeDtypeStruct((_NC, n_pad, dh), jnp.float32),
        mesh=mesh,
        scratch_types=(
            [pltpu.VMEM((_G, _CH), jnp.int32) for _ in range(4)]
            + [pltpu.VMEM((_CH, dh), jnp.float32) for _ in range(4)]
            + [pltpu.VMEM_SHARED((n_pad, dh), jnp.float32),
               pltpu.VMEM_SHARED((n_pad, dh), jnp.float32)]
            + [pltpu.SemaphoreType.DMA for _ in range(10)]
        ),
        compiler_params=pltpu.CompilerParams(use_tc_tiling_on_sc=False),
    )(functools.partial(_prop_kernel_body, n_pad, cpt, dh))

    blk = 1024
    grid = (n_pad // blk,)
    deg_spec = pl.BlockSpec((_NC, 2, blk, _L), lambda i: (0, 0, i, 0))
    w_spec = pl.BlockSpec((d, 1), lambda i: (0, 0))
    b_spec = pl.BlockSpec((1, 1), lambda i: (0, 0))
    nd_spec = pl.BlockSpec((blk, d), lambda i: (i, 0))
    g_spec = pl.BlockSpec((_NC, blk, dh), lambda i: (0, i, 0))

    g, out_acc = pl.pallas_call(
        _combine0_body,
        grid=grid,
        in_specs=[nd_spec, deg_spec, w_spec, b_spec],
        out_specs=[g_spec, nd_spec],
        out_shape=[
            jax.ShapeDtypeStruct((_NC, n_pad, dh), jnp.float32),
            jax.ShapeDtypeStruct((n_pad, d), jnp.float32),
        ],
    )(feat_pad, deg, W, b2)

    combine = pl.pallas_call(
        _combine_body,
        grid=grid,
        in_specs=[g_spec, deg_spec, w_spec, b_spec, nd_spec],
        out_specs=[g_spec, nd_spec],
        out_shape=[
            jax.ShapeDtypeStruct((_NC, n_pad, dh), jnp.float32),
            jax.ShapeDtypeStruct((n_pad, d), jnp.float32),
        ],
        input_output_aliases={4: 1},
    )

    for _ in range(k_rounds):
        accs = prop_call(g, src3, dst3, zbd)
        g, out_acc = combine(accs, deg, W, b2, out_acc)

    return out_acc[:n]


# speedup vs baseline: 1.0813x; 1.0813x over previous
import functools

import jax
import jax.numpy as jnp
from jax import lax
from jax.experimental import pallas as pl
from jax.experimental.pallas import tpu as pltpu
from jax.experimental.pallas import tpu_sc as plsc

_NC = 2
_NS = 16
_NW = _NC * _NS
_L = 16
_CH = 128
_G = 4


def _deg_kernel_body(n_pad, cpt, src_hbm, dst_hbm, ones_hbm, zb_hbm, out_hbm,
                     src_t, dst_t, ones_t, stage_t, din_sh, dout_sh):
    rows_per_tile = n_pad // _NS
    cid = lax.axis_index("c")
    sid = lax.axis_index("s")
    wid = cid * _NS + sid
    pltpu.sync_copy(src_hbm.at[wid], src_t)
    pltpu.sync_copy(dst_hbm.at[wid], dst_t)
    pltpu.sync_copy(ones_hbm, ones_t)
    pltpu.sync_copy(zb_hbm, stage_t)
    r0 = sid * rows_per_tile
    for jj in range(rows_per_tile // _CH):
        pltpu.sync_copy(stage_t, din_sh.at[pl.ds(r0 + jj * _CH, _CH)])
        pltpu.sync_copy(stage_t, dout_sh.at[pl.ds(r0 + jj * _CH, _CH)])
    plsc.subcore_barrier()

    @pl.loop(0, cpt)
    def _(j):
        pltpu.sync_copy(ones_t, din_sh.at[dst_t.at[j]], add=True)
        pltpu.sync_copy(ones_t, dout_sh.at[src_t.at[j]], add=True)

    plsc.subcore_barrier()
    for jj in range(rows_per_tile // _CH):
        sl = pl.ds(r0 + jj * _CH, _CH)
        pltpu.sync_copy(din_sh.at[sl], stage_t)
        pltpu.sync_copy(stage_t, out_hbm.at[cid, 0, sl])
        pltpu.sync_copy(dout_sh.at[sl], stage_t)
        pltpu.sync_copy(stage_t, out_hbm.at[cid, 1, sl])


def _prop_kernel_body(n_pad, cpt, dh, g_hbm, src_hbm, dst_hbm, zb_hbm, out_hbm,
                      sidx0, sidx1, didx0, didx1, rows0, rows1, rows2, rows3,
                      g_sh, acc_sh, gsem0, gsem1, gsem2, gsem3,
                      ssem0, ssem1, ssem2, ssem3, isem0, isem1):
    sidx = (sidx0, sidx1)
    didx = (didx0, didx1)
    rows = (rows0, rows1, rows2, rows3)
    gsem = (gsem0, gsem1, gsem2, gsem3)
    ssem = (ssem0, ssem1, ssem2, ssem3)
    isem = (isem0, isem1)
    rows_per_tile = n_pad // _NS
    cid = lax.axis_index("c")
    sid = lax.axis_index("s")
    r0 = sid * rows_per_tile
    ngrp = cpt // _G

    for jj in range(rows_per_tile // _CH):
        sl = pl.ds(r0 + jj * _CH, _CH)
        pltpu.sync_copy(g_hbm.at[cid, sl], rows0)
        pltpu.sync_copy(rows0, g_sh.at[sl])
    pltpu.sync_copy(zb_hbm, rows1)
    for jj in range(rows_per_tile // _CH):
        pltpu.sync_copy(rows1, acc_sh.at[pl.ds(r0 + jj * _CH, _CH)])
    plsc.subcore_barrier()

    def idx_issue(grp, slot):
        pltpu.async_copy(src_hbm.at[sid, pl.ds(grp * _G, _G)], sidx[slot],
                         isem[slot])
        pltpu.async_copy(dst_hbm.at[sid, pl.ds(grp * _G, _G)], didx[slot],
                         isem[slot])

    def idx_wait(slot):
        pltpu.make_async_copy(src_hbm.at[sid, pl.ds(0, _G)], sidx[slot],
                              isem[slot]).wait()
        pltpu.make_async_copy(dst_hbm.at[sid, pl.ds(0, _G)], didx[slot],
                              isem[slot]).wait()

    idx_issue(0, 0)
    idx_wait(0)
    idx_issue(1, 1)

    @pl.loop(0, ngrp // 2)
    def _(i2):
        g0 = 2 * i2
        chunks = [(0, b) for b in range(_G)] + [(1, b) for b in range(_G)]
        ds = [None] * (2 * _G)
        ss = [None] * (2 * _G)

        def gather(c):
            slot, b = chunks[c]
            ds[c] = pltpu.async_copy(g_sh.at[sidx[slot].at[b]], rows[c % 4],
                                     gsem[c % 4])

        for c in range(3):
            gather(c)
        for c in range(2 * _G):
            if c == 1:
                idx_wait(1)
            slot, b = chunks[c]
            ds[c].wait()
            ss[c] = pltpu.async_copy(rows[c % 4], acc_sh.at[didx[slot].at[b]],
                                     ssem[c % 4], add=True)
            if c + 3 < 2 * _G:
                if c >= 1:
                    ss[c - 1].wait()
                gather(c + 3)
            if c == 4:
                @pl.when(g0 + 2 < ngrp)
                def _():
                    idx_issue(g0 + 2, 0)

        for c in range(2 * _G - 4, 2 * _G):
            ss[c].wait()

        @pl.when(g0 + 2 < ngrp)
        def _():
            idx_wait(0)

            @pl.when(g0 + 3 < ngrp)
            def _():
                idx_issue(g0 + 3, 1)

    plsc.subcore_barrier()
    for jj in range(rows_per_tile // _CH):
        sl = pl.ds(r0 + jj * _CH, _CH)
        pltpu.sync_copy(acc_sh.at[sl], rows0)
        pltpu.sync_copy(rows0, out_hbm.at[cid, sl])


def _combine0_body(feat_r, deg_r, w_r, b_r, g_r, out_r):
    dh = feat_r.shape[1] // 2
    dout = deg_r[0, 1, :, 0:1] + deg_r[1, 1, :, 0:1]
    inv_out = lax.rsqrt(jnp.maximum(dout, 1.0))
    h = feat_r[...]
    s = jnp.sum(h * w_r[:, 0][None, :], axis=1, keepdims=True) + b_r[0, 0]
    out_r[...] = s * h
    g = h * inv_out
    g_r[0] = g[:, :dh]
    g_r[1] = g[:, dh:]


def _combine_body(acc_r, deg_r, w_r, b_r, prev_r, g_r, out_r):
    din = deg_r[0, 0, :, 0:1] + deg_r[1, 0, :, 0:1]
    dout = deg_r[0, 1, :, 0:1] + deg_r[1, 1, :, 0:1]
    inv_in = lax.rsqrt(jnp.maximum(din, 1.0))
    inv_out = lax.rsqrt(jnp.maximum(dout, 1.0))
    h = jnp.concatenate([acc_r[0], acc_r[1]], axis=1) * inv_in
    s = jnp.sum(h * w_r[:, 0][None, :], axis=1, keepdims=True) + b_r[0, 0]
    out_r[...] = prev_r[...] + s * h
    g = h * inv_out
    dh = g.shape[1] // 2
    g_r[0] = g[:, :dh]
    g_r[1] = g[:, dh:]


def kernel(feat, edge_index, W, b):
    n, d = feat.shape
    dh = d // 2
    e = edge_index.shape[1]
    k_rounds = 10

    n_pad = -(-n // (_NS * _CH)) * (_NS * _CH)
    et = e + n
    cpt = -(-et // (_NS * _CH * 2 * _G)) * (2 * _G)
    e_pad = _NS * cpt * _CH

    idx_dtype = edge_index.dtype
    loop = jnp.arange(n, dtype=idx_dtype)
    pad = jnp.full((e_pad - et,), n, dtype=idx_dtype)
    src3 = jnp.concatenate([edge_index[0], loop, pad]).reshape(_NS, cpt, _CH)
    dst3 = jnp.concatenate([edge_index[1], loop, pad]).reshape(_NS, cpt, _CH)
    src_deg = src3.reshape(_NW, cpt // 2, _CH)
    dst_deg = dst3.reshape(_NW, cpt // 2, _CH)
    feat_pad = jnp.zeros((n_pad, d), feat.dtype).at[:n].set(feat)
    ones16 = jnp.ones((_CH, _L), jnp.float32)
    zb16 = jnp.zeros((_CH, _L), jnp.float32)
    zbd = jnp.zeros((_CH, dh), jnp.float32)
    b2 = b.reshape(1, 1)

    mesh = plsc.VectorSubcoreMesh(core_axis_name="c", subcore_axis_name="s")

    deg_call = functools.partial(
        pl.kernel,
        out_type=jax.ShapeDtypeStruct((_NC, 2, n_pad, _L), jnp.float32),
        mesh=mesh,
        scratch_types=[
            pltpu.VMEM((cpt // 2, _CH), jnp.int32),
            pltpu.VMEM((cpt // 2, _CH), jnp.int32),
            pltpu.VMEM((_CH, _L), jnp.float32),
            pltpu.VMEM((_CH, _L), jnp.float32),
            pltpu.VMEM_SHARED((n_pad, _L), jnp.float32),
            pltpu.VMEM_SHARED((n_pad, _L), jnp.float32),
        ],
        compiler_params=pltpu.CompilerParams(use_tc_tiling_on_sc=False),
    )(functools.partial(_deg_kernel_body, n_pad, cpt // 2))
    deg = deg_call(src_deg, dst_deg, ones16, zb16)

    prop_call = functools.partial(
        pl.kernel,
        out_type=jax.ShapeDtypeStruct((_NC, n_pad, dh), jnp.float32),
        mesh=mesh,
        scratch_types=(
            [pltpu.VMEM((_G, _CH), jnp.int32) for _ in range(4)]
            + [pltpu.VMEM((_CH, dh), jnp.float32) for _ in range(4)]
            + [pltpu.VMEM_SHARED((n_pad, dh), jnp.float32),
               pltpu.VMEM_SHARED((n_pad, dh), jnp.float32)]
            + [pltpu.SemaphoreType.DMA for _ in range(10)]
        ),
        compiler_params=pltpu.CompilerParams(use_tc_tiling_on_sc=False),
    )(functools.partial(_prop_kernel_body, n_pad, cpt, dh))

    blk = 1024
    grid = (n_pad // blk,)
    deg_spec = pl.BlockSpec((_NC, 2, blk, _L), lambda i: (0, 0, i, 0))
    w_spec = pl.BlockSpec((d, 1), lambda i: (0, 0))
    b_spec = pl.BlockSpec((1, 1), lambda i: (0, 0))
    nd_spec = pl.BlockSpec((blk, d), lambda i: (i, 0))
    g_spec = pl.BlockSpec((_NC, blk, dh), lambda i: (0, i, 0))

    g, out_acc = pl.pallas_call(
        _combine0_body,
        grid=grid,
        in_specs=[nd_spec, deg_spec, w_spec, b_spec],
        out_specs=[g_spec, nd_spec],
        out_shape=[
            jax.ShapeDtypeStruct((_NC, n_pad, dh), jnp.float32),
            jax.ShapeDtypeStruct((n_pad, d), jnp.float32),
        ],
    )(feat_pad, deg, W, b2)

    combine = pl.pallas_call(
        _combine_body,
        grid=grid,
        in_specs=[g_spec, deg_spec, w_spec, b_spec, nd_spec],
        out_specs=[g_spec, nd_spec],
        out_shape=[
            jax.ShapeDtypeStruct((_NC, n_pad, dh), jnp.float32),
            jax.ShapeDtypeStruct((n_pad, d), jnp.float32),
        ],
        input_output_aliases={4: 1},
    )

    for _ in range(k_rounds):
        accs = prop_call(g, src3, dst3, zbd)
        g, out_acc = combine(accs, deg, W, b2, out_acc)

    return out_acc[:n]


# speedup vs baseline: 1.0816x; 1.0002x over previous
import functools

import jax
import jax.numpy as jnp
from jax import lax
from jax.experimental import pallas as pl
from jax.experimental.pallas import tpu as pltpu
from jax.experimental.pallas import tpu_sc as plsc

_NC = 2
_NS = 16
_NW = _NC * _NS
_L = 16
_CH = 128
_G = 4


def _deg_kernel_body(n_pad, cpt, src_hbm, dst_hbm, ones_hbm, zb_hbm, out_hbm,
                     src_t, dst_t, ones_t, stage_t, din_sh, dout_sh,
                     dsem0, dsem1, dsem2, dsem3):
    rows_per_tile = n_pad // _NS
    cid = lax.axis_index("c")
    sid = lax.axis_index("s")
    wid = cid * _NS + sid
    pltpu.sync_copy(src_hbm.at[wid], src_t)
    pltpu.sync_copy(dst_hbm.at[wid], dst_t)
    pltpu.sync_copy(ones_hbm, ones_t)
    pltpu.sync_copy(zb_hbm, stage_t)
    r0 = sid * rows_per_tile
    for jj in range(rows_per_tile // _CH):
        pltpu.sync_copy(stage_t, din_sh.at[pl.ds(r0 + jj * _CH, _CH)])
        pltpu.sync_copy(stage_t, dout_sh.at[pl.ds(r0 + jj * _CH, _CH)])
    plsc.subcore_barrier()

    @pl.loop(0, cpt // 2)
    def _(i):
        j0 = 2 * i
        a0 = pltpu.async_copy(ones_t, din_sh.at[dst_t.at[j0]], dsem0,
                              add=True)
        a1 = pltpu.async_copy(ones_t, dout_sh.at[src_t.at[j0]], dsem1,
                              add=True)
        a2 = pltpu.async_copy(ones_t, din_sh.at[dst_t.at[j0 + 1]], dsem2,
                              add=True)
        a3 = pltpu.async_copy(ones_t, dout_sh.at[src_t.at[j0 + 1]], dsem3,
                              add=True)
        a0.wait()
        a1.wait()
        a2.wait()
        a3.wait()

    plsc.subcore_barrier()
    for jj in range(rows_per_tile // _CH):
        sl = pl.ds(r0 + jj * _CH, _CH)
        pltpu.sync_copy(din_sh.at[sl], stage_t)
        pltpu.sync_copy(stage_t, out_hbm.at[cid, 0, sl])
        pltpu.sync_copy(dout_sh.at[sl], stage_t)
        pltpu.sync_copy(stage_t, out_hbm.at[cid, 1, sl])


def _prop_kernel_body(n_pad, cpt, dh, g_hbm, src_hbm, dst_hbm, zb_hbm, out_hbm,
                      sidx0, sidx1, didx0, didx1, rows0, rows1, rows2, rows3,
                      g_sh, acc_sh, gsem0, gsem1, gsem2, gsem3,
                      ssem0, ssem1, ssem2, ssem3, isem0, isem1):
    sidx = (sidx0, sidx1)
    didx = (didx0, didx1)
    rows = (rows0, rows1, rows2, rows3)
    gsem = (gsem0, gsem1, gsem2, gsem3)
    ssem = (ssem0, ssem1, ssem2, ssem3)
    isem = (isem0, isem1)
    rows_per_tile = n_pad // _NS
    cid = lax.axis_index("c")
    sid = lax.axis_index("s")
    r0 = sid * rows_per_tile
    ngrp = cpt // _G

    for jj in range(rows_per_tile // _CH):
        sl = pl.ds(r0 + jj * _CH, _CH)
        pltpu.sync_copy(g_hbm.at[cid, sl], rows0)
        pltpu.sync_copy(rows0, g_sh.at[sl])
    pltpu.sync_copy(zb_hbm, rows1)
    for jj in range(rows_per_tile // _CH):
        pltpu.sync_copy(rows1, acc_sh.at[pl.ds(r0 + jj * _CH, _CH)])
    plsc.subcore_barrier()

    def idx_issue(grp, slot):
        pltpu.async_copy(src_hbm.at[sid, pl.ds(grp * _G, _G)], sidx[slot],
                         isem[slot])
        pltpu.async_copy(dst_hbm.at[sid, pl.ds(grp * _G, _G)], didx[slot],
                         isem[slot])

    def idx_wait(slot):
        pltpu.make_async_copy(src_hbm.at[sid, pl.ds(0, _G)], sidx[slot],
                              isem[slot]).wait()
        pltpu.make_async_copy(dst_hbm.at[sid, pl.ds(0, _G)], didx[slot],
                              isem[slot]).wait()

    idx_issue(0, 0)
    idx_wait(0)
    idx_issue(1, 1)

    @pl.loop(0, ngrp // 2)
    def _(i2):
        g0 = 2 * i2
        chunks = [(0, b) for b in range(_G)] + [(1, b) for b in range(_G)]
        ds = [None] * (2 * _G)
        ss = [None] * (2 * _G)

        def gather(c):
            slot, b = chunks[c]
            ds[c] = pltpu.async_copy(g_sh.at[sidx[slot].at[b]], rows[c % 4],
                                     gsem[c % 4])

        for c in range(3):
            gather(c)
        for c in range(2 * _G):
            if c == 1:
                idx_wait(1)
            slot, b = chunks[c]
            ds[c].wait()
            ss[c] = pltpu.async_copy(rows[c % 4], acc_sh.at[didx[slot].at[b]],
                                     ssem[c % 4], add=True)
            if c + 3 < 2 * _G:
                if c >= 1:
                    ss[c - 1].wait()
                gather(c + 3)
            if c == 4:
                @pl.when(g0 + 2 < ngrp)
                def _():
                    idx_issue(g0 + 2, 0)

        for c in range(2 * _G - 4, 2 * _G):
            ss[c].wait()

        @pl.when(g0 + 2 < ngrp)
        def _():
            idx_wait(0)

            @pl.when(g0 + 3 < ngrp)
            def _():
                idx_issue(g0 + 3, 1)

    plsc.subcore_barrier()
    for jj in range(rows_per_tile // _CH):
        sl = pl.ds(r0 + jj * _CH, _CH)
        pltpu.sync_copy(acc_sh.at[sl], rows0)
        pltpu.sync_copy(rows0, out_hbm.at[cid, sl])


def _combine0_body(feat_r, deg_r, w_r, b_r, g_r, out_r):
    dh = feat_r.shape[1] // 2
    dout = deg_r[0, 1, :, 0:1] + deg_r[1, 1, :, 0:1]
    inv_out = lax.rsqrt(jnp.maximum(dout, 1.0))
    h = feat_r[...]
    s = jnp.sum(h * w_r[:, 0][None, :], axis=1, keepdims=True) + b_r[0, 0]
    out_r[...] = s * h
    g = h * inv_out
    g_r[0] = g[:, :dh]
    g_r[1] = g[:, dh:]


def _combine_body(acc_r, deg_r, w_r, b_r, prev_r, g_r, out_r):
    din = deg_r[0, 0, :, 0:1] + deg_r[1, 0, :, 0:1]
    dout = deg_r[0, 1, :, 0:1] + deg_r[1, 1, :, 0:1]
    inv_in = lax.rsqrt(jnp.maximum(din, 1.0))
    inv_out = lax.rsqrt(jnp.maximum(dout, 1.0))
    h = jnp.concatenate([acc_r[0], acc_r[1]], axis=1) * inv_in
    s = jnp.sum(h * w_r[:, 0][None, :], axis=1, keepdims=True) + b_r[0, 0]
    out_r[...] = prev_r[...] + s * h
    g = h * inv_out
    dh = g.shape[1] // 2
    g_r[0] = g[:, :dh]
    g_r[1] = g[:, dh:]


def kernel(feat, edge_index, W, b):
    n, d = feat.shape
    dh = d // 2
    e = edge_index.shape[1]
    k_rounds = 10

    n_pad = -(-n // (_NS * _CH)) * (_NS * _CH)
    et = e + n
    cpt = -(-et // (_NS * _CH * 2 * _G)) * (2 * _G)
    e_pad = _NS * cpt * _CH

    idx_dtype = edge_index.dtype
    loop = jnp.arange(n, dtype=idx_dtype)
    pad = jnp.full((e_pad - et,), n, dtype=idx_dtype)
    src3 = jnp.concatenate([edge_index[0], loop, pad]).reshape(_NS, cpt, _CH)
    dst3 = jnp.concatenate([edge_index[1], loop, pad]).reshape(_NS, cpt, _CH)
    src_deg = src3.reshape(_NW, cpt // 2, _CH)
    dst_deg = dst3.reshape(_NW, cpt // 2, _CH)
    feat_pad = jnp.zeros((n_pad, d), feat.dtype).at[:n].set(feat)
    ones16 = jnp.ones((_CH, _L), jnp.float32)
    zb16 = jnp.zeros((_CH, _L), jnp.float32)
    zbd = jnp.zeros((_CH, dh), jnp.float32)
    b2 = b.reshape(1, 1)

    mesh = plsc.VectorSubcoreMesh(core_axis_name="c", subcore_axis_name="s")

    deg_call = functools.partial(
        pl.kernel,
        out_type=jax.ShapeDtypeStruct((_NC, 2, n_pad, _L), jnp.float32),
        mesh=mesh,
        scratch_types=[
            pltpu.VMEM((cpt // 2, _CH), jnp.int32),
            pltpu.VMEM((cpt // 2, _CH), jnp.int32),
            pltpu.VMEM((_CH, _L), jnp.float32),
            pltpu.VMEM((_CH, _L), jnp.float32),
            pltpu.VMEM_SHARED((n_pad, _L), jnp.float32),
            pltpu.VMEM_SHARED((n_pad, _L), jnp.float32),
            pltpu.SemaphoreType.DMA,
            pltpu.SemaphoreType.DMA,
            pltpu.SemaphoreType.DMA,
            pltpu.SemaphoreType.DMA,
        ],
        compiler_params=pltpu.CompilerParams(use_tc_tiling_on_sc=False),
    )(functools.partial(_deg_kernel_body, n_pad, cpt // 2))
    deg = deg_call(src_deg, dst_deg, ones16, zb16)

    prop_call = functools.partial(
        pl.kernel,
        out_type=jax.ShapeDtypeStruct((_NC, n_pad, dh), jnp.float32),
        mesh=mesh,
        scratch_types=(
            [pltpu.VMEM((_G, _CH), jnp.int32) for _ in range(4)]
            + [pltpu.VMEM((_CH, dh), jnp.float32) for _ in range(4)]
            + [pltpu.VMEM_SHARED((n_pad, dh), jnp.float32),
               pltpu.VMEM_SHARED((n_pad, dh), jnp.float32)]
            + [pltpu.SemaphoreType.DMA for _ in range(10)]
        ),
        compiler_params=pltpu.CompilerParams(use_tc_tiling_on_sc=False),
    )(functools.partial(_prop_kernel_body, n_pad, cpt, dh))

    blk = 1024
    grid = (n_pad // blk,)
    deg_spec = pl.BlockSpec((_NC, 2, blk, _L), lambda i: (0, 0, i, 0))
    w_spec = pl.BlockSpec((d, 1), lambda i: (0, 0))
    b_spec = pl.BlockSpec((1, 1), lambda i: (0, 0))
    nd_spec = pl.BlockSpec((blk, d), lambda i: (i, 0))
    g_spec = pl.BlockSpec((_NC, blk, dh), lambda i: (0, i, 0))

    g, out_acc = pl.pallas_call(
        _combine0_body,
        grid=grid,
        in_specs=[nd_spec, deg_spec, w_spec, b_spec],
        out_specs=[g_spec, nd_spec],
        out_shape=[
            jax.ShapeDtypeStruct((_NC, n_pad, dh), jnp.float32),
            jax.ShapeDtypeStruct((n_pad, d), jnp.float32),
        ],
    )(feat_pad, deg, W, b2)

    combine = pl.pallas_call(
        _combine_body,
        grid=grid,
        in_specs=[g_spec, deg_spec, w_spec, b_spec, nd_spec],
        out_specs=[g_spec, nd_spec],
        out_shape=[
            jax.ShapeDtypeStruct((_NC, n_pad, dh), jnp.float32),
            jax.ShapeDtypeStruct((n_pad, d), jnp.float32),
        ],
        input_output_aliases={4: 1},
    )

    for _ in range(k_rounds):
        accs = prop_call(g, src3, dst3, zbd)
        g, out_acc = combine(accs, deg, W, b2, out_acc)

    return out_acc[:n]
